# baseline (speedup 1.0000x reference)
import functools

import jax
import jax.numpy as jnp
from jax import lax
from jax.experimental import pallas as pl
from jax.experimental.pallas import tpu as pltpu
from jax.experimental.pallas import tpu_sc as plsc

B, N, D, L, V, PAD, R = 8, 512, 128, 4, 32, 0, 6.0
BN = B * N
NT = 32
RPT = BN // NT
CAP = 8192

_HI = jax.lax.Precision.HIGHEST


def _dgT(a, b):
    return jax.lax.dot_general(a, b, (((1,), (1,)), ((), ())), precision=_HI)


def _mm(a, b):
    return jax.lax.dot_general(a, b, (((1,), (0,)), ((), ())), precision=_HI)


def _tc0_body(oh_ref, cpad_ref, tok_ref, cx_ref, cy_ref, cz_ref, embed_ref,
              w1_ref, b1_ref, be_ref, we_ref,
              md_ref, x0_ref, y0_ref, deg_ref, u_ref):
    oh = oh_ref[...]
    C = cpad_ref[...]
    CC = C * C
    ones = jnp.ones((N, D), jnp.float32)
    cxr, cyr, czr = cx_ref[0], cy_ref[0], cz_ref[0]
    sqi = _dgT(CC, ones)
    sqj = cxr * cxr + cyr * cyr + czr * czr
    dots = jax.lax.dot_general(C, C, (((1,), (1,)), ((), ())),
                               precision=jax.lax.Precision.DEFAULT)
    d2 = sqi + sqj - 2.0 * dots
    ri = jax.lax.broadcasted_iota(jnp.int32, (N, N), 0)
    rj = jax.lax.broadcasted_iota(jnp.int32, (N, N), 1)
    vi = oh[:, 0:1] < 0.5
    vj = tok_ref[0] != PAD
    adj = (d2 < R * R) & (ri != rj) & vi & vj
    ddx = C[:, 0:1] - cxr
    ddy = C[:, 1:2] - cyr
    ddz = C[:, 2:3] - czr
    dist = jnp.sqrt(ddx * ddx + ddy * ddy + ddz * ddz)
    md_ref[...] = jnp.where(adj, dist, -1.0)
    adjf = adj.astype(jnp.float32)
    deg_ref[...] = _mm(adjf, ones)
    x0 = _mm(oh, embed_ref[...])
    x0_ref[...] = x0
    A = w1_ref[0:D, :]
    Bm = w1_ref[D:2 * D, :]
    cvec = _mm(be_ref[...], Bm) + b1_ref[...]
    y0_ref[...] = _mm(x0, A) + cvec
    u_ref[...] = _mm(we_ref[...], Bm)


def _tc0(oh, cpad, tok, cx, cy, cz, embed, w1, b1, be, we):
    blk = lambda shape, imap: pl.BlockSpec(shape, imap)
    return pl.pallas_call(
        _tc0_body,
        grid=(B,),
        in_specs=[
            blk((N, V), lambda b: (b, 0)),
            blk((N, D), lambda b: (b, 0)),
            blk((1, 1, N), lambda b: (b, 0, 0)),
            blk((1, 1, N), lambda b: (b, 0, 0)),
            blk((1, 1, N), lambda b: (b, 0, 0)),
            blk((1, 1, N), lambda b: (b, 0, 0)),
            blk((V, D), lambda b: (0, 0)),
            blk((2 * D, D), lambda b: (0, 0)),
            blk((1, D), lambda b: (0, 0)),
            blk((1, D), lambda b: (0, 0)),
            blk((1, D), lambda b: (0, 0)),
        ],
        out_specs=[
            blk((N, N), lambda b: (b, 0)),
            blk((N, D), lambda b: (b, 0)),
            blk((N, D), lambda b: (b, 0)),
            blk((N, D), lambda b: (b, 0)),
            blk((1, D), lambda b: (0, 0)),
        ],
        out_shape=[
            jax.ShapeDtypeStruct((BN, N), jnp.float32),
            jax.ShapeDtypeStruct((BN, D), jnp.float32),
            jax.ShapeDtypeStruct((BN, D), jnp.float32),
            jax.ShapeDtypeStruct((BN, D), jnp.float32),
            jax.ShapeDtypeStruct((1, D), jnp.float32),
        ],
    )(oh, cpad, tok, cx, cy, cz, embed, w1, b1, be, we)


def _tc_mid_body(x_ref, h_ref, deg_ref, w2_ref, b2_ref, w1_ref, b1_ref,
                 be_ref, we_ref, xn_ref, y_ref, u_ref):
    xn = x_ref[...] + _mm(h_ref[...], w2_ref[...]) + deg_ref[...] * b2_ref[...]
    xn_ref[...] = xn
    A = w1_ref[0:D, :]
    Bm = w1_ref[D:2 * D, :]
    cvec = _mm(be_ref[...], Bm) + b1_ref[...]
    y_ref[...] = _mm(xn, A) + cvec
    u_ref[...] = _mm(we_ref[...], Bm)


def _tc_mid(x, h, deg, w2, b2, w1, b1, be, we):
    blk = lambda shape, imap: pl.BlockSpec(shape, imap)
    return pl.pallas_call(
        _tc_mid_body,
        grid=(1,),
        in_specs=[
            blk((BN, D), lambda b: (0, 0)),
            blk((BN, D), lambda b: (0, 0)),
            blk((BN, D), lambda b: (0, 0)),
            blk((D, D), lambda b: (0, 0)),
            blk((1, D), lambda b: (0, 0)),
            blk((2 * D, D), lambda b: (0, 0)),
            blk((1, D), lambda b: (0, 0)),
            blk((1, D), lambda b: (0, 0)),
            blk((1, D), lambda b: (0, 0)),
        ],
        out_specs=[
            blk((BN, D), lambda b: (0, 0)),
            blk((BN, D), lambda b: (0, 0)),
            blk((1, D), lambda b: (0, 0)),
        ],
        out_shape=[
            jax.ShapeDtypeStruct((BN, D), jnp.float32),
            jax.ShapeDtypeStruct((BN, D), jnp.float32),
            jax.ShapeDtypeStruct((1, D), jnp.float32),
        ],
    )(x, h, deg, w2, b2, w1, b1, be, we)


def _tc_fin_body(x_ref, h_ref, deg_ref, w2_ref, b2_ref, oh_ref, out_ref):
    xn = x_ref[...] + _mm(h_ref[...], w2_ref[...]) + deg_ref[...] * b2_ref[...]
    vi = oh_ref[:, 0:1] < 0.5
    out_ref[...] = jnp.where(vi, xn, 0.0)


def _tc_fin(x, h, deg, w2, b2, oh):
    blk = lambda shape, imap: pl.BlockSpec(shape, imap)
    return pl.pallas_call(
        _tc_fin_body,
        grid=(1,),
        in_specs=[
            blk((BN, D), lambda b: (0, 0)),
            blk((BN, D), lambda b: (0, 0)),
            blk((BN, D), lambda b: (0, 0)),
            blk((D, D), lambda b: (0, 0)),
            blk((1, D), lambda b: (0, 0)),
            blk((BN, V), lambda b: (0, 0)),
        ],
        out_specs=[blk((BN, D), lambda b: (0, 0))],
        out_shape=[jax.ShapeDtypeStruct((BN, D), jnp.float32)],
    )(x, h, deg, w2, b2, oh)[0]


_MESH = plsc.VectorSubcoreMesh(core_axis_name="c", subcore_axis_name="s")


def _iota16():
    return jax.lax.broadcasted_iota(jnp.int32, (16,), 0)


@functools.partial(
    pl.kernel,
    out_type=[
        jax.ShapeDtypeStruct((NT, CAP), jnp.int32),
        jax.ShapeDtypeStruct((NT, CAP), jnp.int32),
        jax.ShapeDtypeStruct((NT, CAP), jnp.float32),
        jax.ShapeDtypeStruct((NT, 16), jnp.int32),
    ],
    mesh=_MESH,
    compiler_params=pltpu.CompilerParams(needs_layout_passes=False),
    scratch_types=[
        pltpu.VMEM((RPT, N), jnp.float32),
        pltpu.VMEM((CAP,), jnp.int32),
        pltpu.VMEM((CAP,), jnp.int32),
        pltpu.VMEM((CAP,), jnp.float32),
        pltpu.VMEM((16,), jnp.int32),
    ],
)
def _sc_compact(md_hbm, cols_hbm, rows_hbm, dists_hbm, counts_hbm,
                mdv, colv, rowv, distv, cntv):
    c = lax.axis_index("c")
    s = lax.axis_index("s")
    t = c * 16 + s
    it16 = _iota16()
    pltpu.sync_copy(md_hbm.at[pl.ds(t * RPT, RPT)], mdv)

    def row_body(i, ptr):
        isp = jnp.full((16,), i, jnp.int32)

        @plsc.parallel_loop(0, N // 16, carry=ptr, unroll=8)
        def grp_body(g, ptr):
            jloc = g * 16 + it16
            v = plsc.load_gather(mdv, [isp, jloc])
            m = v >= 0.0
            pos = ptr + jnp.cumsum(m.astype(jnp.int32)) - 1
            mm = m & (pos < CAP)
            plsc.store_scatter(colv, [pos], jloc, mask=mm)
            plsc.store_scatter(rowv, [pos], isp, mask=mm)
            plsc.store_scatter(distv, [pos], v, mask=mm)
            return ptr + plsc.all_reduce_population_count(m)
        return grp_body

    ptr = lax.fori_loop(0, RPT, row_body, jnp.zeros((16,), jnp.int32))
    cntv[...] = jnp.minimum(ptr, CAP)
    pltpu.sync_copy(colv, cols_hbm.at[t])
    pltpu.sync_copy(rowv, rows_hbm.at[t])
    pltpu.sync_copy(distv, dists_hbm.at[t])
    pltpu.sync_copy(cntv, counts_hbm.at[t])


@functools.partial(
    pl.kernel,
    out_type=jax.ShapeDtypeStruct((BN, D), jnp.float32),
    mesh=_MESH,
    compiler_params=pltpu.CompilerParams(needs_layout_passes=False),
    scratch_types=[
        pltpu.VMEM((N, D), jnp.float32),
        pltpu.VMEM((RPT, D), jnp.float32),
        pltpu.VMEM((CAP,), jnp.int32),
        pltpu.VMEM((CAP,), jnp.int32),
        pltpu.VMEM((CAP,), jnp.float32),
        pltpu.VMEM((16,), jnp.int32),
        pltpu.VMEM((1, D), jnp.float32),
    ],
)
def _sc_edge(y_hbm, u_hbm, cols_hbm, rows_hbm, dists_hbm, counts_hbm,
             h_hbm, yv, hloc, cbuf, rbuf, dbuf, cntv, uv):
    c = lax.axis_index("c")
    s = lax.axis_index("s")
    t = c * 16 + s
    b = t >> 2
    it16 = _iota16()
    z16 = jnp.zeros((16,), jnp.int32)
    z16f = jnp.zeros((16,), jnp.float32)
    pltpu.sync_copy(y_hbm.at[pl.ds(b * N, N)], yv)
    pltpu.sync_copy(u_hbm, uv)
    pltpu.sync_copy(cols_hbm.at[t], cbuf)
    pltpu.sync_copy(rows_hbm.at[t], rbuf)
    pltpu.sync_copy(dists_hbm.at[t], dbuf)
    pltpu.sync_copy(counts_hbm.at[t], cntv)
    count = jnp.max(cntv[...])

    uks = [plsc.load_gather(uv, [z16, k * 16 + it16])
           for k in range(D // 16)]
    fks = [k * 16 + it16 for k in range(D // 16)]

    @plsc.parallel_loop(0, RPT)
    def zero_body(r):
        rsp = jnp.full((16,), r, jnp.int32)
        for k in range(D // 16):
            plsc.store_scatter(hloc, [rsp, fks[k]], z16f)

    @plsc.parallel_loop(0, count, unroll=4)
    def e_body(e):
        esp = jnp.full((16,), e, jnp.int32)
        colsp = plsc.load_gather(cbuf, [esp])
        rowsp = plsc.load_gather(rbuf, [esp])
        dsp = plsc.load_gather(dbuf, [esp])
        for k in range(D // 16):
            yk = plsc.load_gather(yv, [colsp, fks[k]])
            hk = jnp.maximum(yk + dsp * uks[k], 0.0)
            plsc.addupdate_scatter(hloc, [rowsp, fks[k]], hk)

    pltpu.sync_copy(hloc, h_hbm.at[pl.ds(t * RPT, RPT)])


def kernel(src_tokens, padded_coordinates, src_distance, src_edge_type,
           embed_tokens, params):
    del src_distance, src_edge_type
    tok = src_tokens.astype(jnp.int32)
    coord = padded_coordinates.astype(jnp.float32).reshape(BN, 3)
    cpad = jnp.pad(coord, ((0, 0), (0, D - 3)))
    oh = (tok.reshape(BN, 1) == jnp.arange(V, dtype=jnp.int32)[None, :]
          ).astype(jnp.float32)
    embed = embed_tokens.astype(jnp.float32)

    def prep(p):
        return (p["W1"], p["b1"].reshape(1, D), p["be"].reshape(1, D),
                p["We"].reshape(1, D), p["W2"], p["b2"].reshape(1, D))

    w1_0, b1_0, be_0, we_0, _, _ = prep(params[0])
    c3 = coord.reshape(B, 1, N, 3)
    md, x, y, deg, u = _tc0(oh, cpad, tok.reshape(B, 1, N),
                            c3[..., 0], c3[..., 1], c3[..., 2], embed,
                            w1_0, b1_0, be_0, we_0)
    cols, rows, dists, counts = _sc_compact(md)

    for l in range(L):
        h = _sc_edge(y, u, cols, rows, dists, counts)
        _, _, _, _, w2, b2 = prep(params[l])
        if l + 1 < L:
            w1n, b1n, ben, wen, _, _ = prep(params[l + 1])
            x, y, u = _tc_mid(x, h, deg, w2, b2, w1n, b1n, ben, wen)
        else:
            out = _tc_fin(x, h, deg, w2, b2, oh)

    encoder_rep = out.reshape(B, N, D)
    padding_mask = src_tokens == PAD
    return (encoder_rep, padding_mask)


# speedup vs baseline: 1.0521x; 1.0521x over previous
import functools

import jax
import jax.numpy as jnp
from jax import lax
from jax.experimental import pallas as pl
from jax.experimental.pallas import tpu as pltpu
from jax.experimental.pallas import tpu_sc as plsc

B, N, D, L, V, PAD, R = 8, 512, 128, 4, 32, 0, 6.0
BN = B * N
NT = 32
RPT = BN // NT
CAP = 8192

_HI = jax.lax.Precision.HIGHEST


def _dgT(a, b):
    return jax.lax.dot_general(a, b, (((1,), (1,)), ((), ())), precision=_HI)


def _mm(a, b):
    return jax.lax.dot_general(a, b, (((1,), (0,)), ((), ())), precision=_HI)


def _tc0_body(oh_ref, cpad_ref, tok_ref, cx_ref, cy_ref, cz_ref, embed_ref,
              w1_ref, b1_ref, be_ref, we_ref,
              md_ref, x0_ref, y0_ref, deg_ref, u_ref):
    oh = oh_ref[...]
    C = cpad_ref[...]
    CC = C * C
    ones = jnp.ones((N, D), jnp.float32)
    cxr, cyr, czr = cx_ref[0], cy_ref[0], cz_ref[0]
    sqi = _dgT(CC, ones)
    sqj = cxr * cxr + cyr * cyr + czr * czr
    dots = jax.lax.dot_general(C, C, (((1,), (1,)), ((), ())),
                               precision=jax.lax.Precision.DEFAULT)
    d2 = sqi + sqj - 2.0 * dots
    ri = jax.lax.broadcasted_iota(jnp.int32, (N, N), 0)
    rj = jax.lax.broadcasted_iota(jnp.int32, (N, N), 1)
    vi = oh[:, 0:1] < 0.5
    vj = tok_ref[0] != PAD
    adj = (d2 < R * R) & (ri != rj) & vi & vj
    ddx = C[:, 0:1] - cxr
    ddy = C[:, 1:2] - cyr
    ddz = C[:, 2:3] - czr
    dist = jnp.sqrt(ddx * ddx + ddy * ddy + ddz * ddz)
    md_ref[...] = jnp.where(adj, dist, -1.0)
    adjf = adj.astype(jnp.float32)
    deg_ref[...] = _mm(adjf, ones)
    x0 = _mm(oh, embed_ref[...])
    x0_ref[...] = x0
    A = w1_ref[0:D, :]
    Bm = w1_ref[D:2 * D, :]
    cvec = _mm(be_ref[...], Bm) + b1_ref[...]
    y0_ref[...] = _mm(x0, A) + cvec
    u_ref[...] = _mm(we_ref[...], Bm)


def _tc0(oh, cpad, tok, cx, cy, cz, embed, w1, b1, be, we):
    blk = lambda shape, imap: pl.BlockSpec(shape, imap)
    return pl.pallas_call(
        _tc0_body,
        grid=(B,),
        in_specs=[
            blk((N, V), lambda b: (b, 0)),
            blk((N, D), lambda b: (b, 0)),
            blk((1, 1, N), lambda b: (b, 0, 0)),
            blk((1, 1, N), lambda b: (b, 0, 0)),
            blk((1, 1, N), lambda b: (b, 0, 0)),
            blk((1, 1, N), lambda b: (b, 0, 0)),
            blk((V, D), lambda b: (0, 0)),
            blk((2 * D, D), lambda b: (0, 0)),
            blk((1, D), lambda b: (0, 0)),
            blk((1, D), lambda b: (0, 0)),
            blk((1, D), lambda b: (0, 0)),
        ],
        out_specs=[
            blk((N, N), lambda b: (b, 0)),
            blk((N, D), lambda b: (b, 0)),
            blk((N, D), lambda b: (b, 0)),
            blk((N, D), lambda b: (b, 0)),
            blk((1, D), lambda b: (0, 0)),
        ],
        out_shape=[
            jax.ShapeDtypeStruct((BN, N), jnp.float32),
            jax.ShapeDtypeStruct((BN, D), jnp.float32),
            jax.ShapeDtypeStruct((BN, D), jnp.float32),
            jax.ShapeDtypeStruct((BN, D), jnp.float32),
            jax.ShapeDtypeStruct((1, D), jnp.float32),
        ],
    )(oh, cpad, tok, cx, cy, cz, embed, w1, b1, be, we)


def _tc_mid_body(x_ref, h_ref, deg_ref, w2_ref, b2_ref, w1_ref, b1_ref,
                 be_ref, we_ref, xn_ref, y_ref, u_ref):
    xn = x_ref[...] + _mm(h_ref[...], w2_ref[...]) + deg_ref[...] * b2_ref[...]
    xn_ref[...] = xn
    A = w1_ref[0:D, :]
    Bm = w1_ref[D:2 * D, :]
    cvec = _mm(be_ref[...], Bm) + b1_ref[...]
    y_ref[...] = _mm(xn, A) + cvec
    u_ref[...] = _mm(we_ref[...], Bm)


def _tc_mid(x, h, deg, w2, b2, w1, b1, be, we):
    blk = lambda shape, imap: pl.BlockSpec(shape, imap)
    return pl.pallas_call(
        _tc_mid_body,
        grid=(B,),
        in_specs=[
            blk((N, D), lambda b: (b, 0)),
            blk((N, D), lambda b: (b, 0)),
            blk((N, D), lambda b: (b, 0)),
            blk((D, D), lambda b: (0, 0)),
            blk((1, D), lambda b: (0, 0)),
            blk((2 * D, D), lambda b: (0, 0)),
            blk((1, D), lambda b: (0, 0)),
            blk((1, D), lambda b: (0, 0)),
            blk((1, D), lambda b: (0, 0)),
        ],
        out_specs=[
            blk((N, D), lambda b: (b, 0)),
            blk((N, D), lambda b: (b, 0)),
            blk((1, D), lambda b: (0, 0)),
        ],
        out_shape=[
            jax.ShapeDtypeStruct((BN, D), jnp.float32),
            jax.ShapeDtypeStruct((BN, D), jnp.float32),
            jax.ShapeDtypeStruct((1, D), jnp.float32),
        ],
    )(x, h, deg, w2, b2, w1, b1, be, we)


def _tc_fin_body(x_ref, h_ref, deg_ref, w2_ref, b2_ref, oh_ref, out_ref):
    xn = x_ref[...] + _mm(h_ref[...], w2_ref[...]) + deg_ref[...] * b2_ref[...]
    vi = oh_ref[:, 0:1] < 0.5
    out_ref[...] = jnp.where(vi, xn, 0.0)


def _tc_fin(x, h, deg, w2, b2, oh):
    blk = lambda shape, imap: pl.BlockSpec(shape, imap)
    return pl.pallas_call(
        _tc_fin_body,
        grid=(B,),
        in_specs=[
            blk((N, D), lambda b: (b, 0)),
            blk((N, D), lambda b: (b, 0)),
            blk((N, D), lambda b: (b, 0)),
            blk((D, D), lambda b: (0, 0)),
            blk((1, D), lambda b: (0, 0)),
            blk((N, V), lambda b: (b, 0)),
        ],
        out_specs=[blk((N, D), lambda b: (b, 0))],
        out_shape=[jax.ShapeDtypeStruct((BN, D), jnp.float32)],
    )(x, h, deg, w2, b2, oh)[0]


_MESH = plsc.VectorSubcoreMesh(core_axis_name="c", subcore_axis_name="s")


def _iota16():
    return jax.lax.broadcasted_iota(jnp.int32, (16,), 0)


@functools.partial(
    pl.kernel,
    out_type=[
        jax.ShapeDtypeStruct((NT, CAP), jnp.int32),
        jax.ShapeDtypeStruct((NT, CAP), jnp.int32),
        jax.ShapeDtypeStruct((NT, CAP), jnp.float32),
        jax.ShapeDtypeStruct((NT, 16), jnp.int32),
    ],
    mesh=_MESH,
    compiler_params=pltpu.CompilerParams(needs_layout_passes=False),
    scratch_types=[
        pltpu.VMEM((RPT, N), jnp.float32),
        pltpu.VMEM((CAP,), jnp.int32),
        pltpu.VMEM((CAP,), jnp.int32),
        pltpu.VMEM((CAP,), jnp.float32),
        pltpu.VMEM((16,), jnp.int32),
    ],
)
def _sc_compact(md_hbm, cols_hbm, rows_hbm, dists_hbm, counts_hbm,
                mdv, colv, rowv, distv, cntv):
    c = lax.axis_index("c")
    s = lax.axis_index("s")
    t = c * 16 + s
    it16 = _iota16()
    pltpu.sync_copy(md_hbm.at[pl.ds(t * RPT, RPT)], mdv)

    def row_body(i, ptr):
        isp = jnp.full((16,), i, jnp.int32)

        @plsc.parallel_loop(0, N // 16, carry=ptr, unroll=8)
        def grp_body(g, ptr):
            jloc = g * 16 + it16
            v = plsc.load_gather(mdv, [isp, jloc])
            m = v >= 0.0
            pos = ptr + jnp.cumsum(m.astype(jnp.int32)) - 1
            mm = m & (pos < CAP)
            plsc.store_scatter(colv, [pos], jloc, mask=mm)
            plsc.store_scatter(rowv, [pos], isp, mask=mm)
            plsc.store_scatter(distv, [pos], v, mask=mm)
            return ptr + plsc.all_reduce_population_count(m)
        return grp_body

    ptr = lax.fori_loop(0, RPT, row_body, jnp.zeros((16,), jnp.int32))
    cntv[...] = jnp.minimum(ptr, CAP)
    pltpu.sync_copy(colv, cols_hbm.at[t])
    pltpu.sync_copy(rowv, rows_hbm.at[t])
    pltpu.sync_copy(distv, dists_hbm.at[t])
    pltpu.sync_copy(cntv, counts_hbm.at[t])


@functools.partial(
    pl.kernel,
    out_type=jax.ShapeDtypeStruct((BN, D), jnp.float32),
    mesh=_MESH,
    compiler_params=pltpu.CompilerParams(needs_layout_passes=False),
    scratch_types=[
        pltpu.VMEM((N, D), jnp.float32),
        pltpu.VMEM((RPT, D), jnp.float32),
        pltpu.VMEM((CAP,), jnp.int32),
        pltpu.VMEM((CAP,), jnp.int32),
        pltpu.VMEM((CAP,), jnp.float32),
        pltpu.VMEM((16,), jnp.int32),
        pltpu.VMEM((1, D), jnp.float32),
    ],
)
def _sc_edge(y_hbm, u_hbm, cols_hbm, rows_hbm, dists_hbm, counts_hbm,
             h_hbm, yv, hloc, cbuf, rbuf, dbuf, cntv, uv):
    c = lax.axis_index("c")
    s = lax.axis_index("s")
    t = c * 16 + s
    b = t >> 2
    it16 = _iota16()
    z16 = jnp.zeros((16,), jnp.int32)
    z16f = jnp.zeros((16,), jnp.float32)
    pltpu.sync_copy(y_hbm.at[pl.ds(b * N, N)], yv)
    pltpu.sync_copy(u_hbm, uv)
    pltpu.sync_copy(cols_hbm.at[t], cbuf)
    pltpu.sync_copy(rows_hbm.at[t], rbuf)
    pltpu.sync_copy(dists_hbm.at[t], dbuf)
    pltpu.sync_copy(counts_hbm.at[t], cntv)
    count = jnp.max(cntv[...])

    uks = [plsc.load_gather(uv, [z16, k * 16 + it16])
           for k in range(D // 16)]
    fks = [k * 16 + it16 for k in range(D // 16)]

    @plsc.parallel_loop(0, RPT)
    def zero_body(r):
        rsp = jnp.full((16,), r, jnp.int32)
        for k in range(D // 16):
            plsc.store_scatter(hloc, [rsp, fks[k]], z16f)

    @plsc.parallel_loop(0, count, unroll=2)
    def e_body(e):
        esp = jnp.full((16,), e, jnp.int32)
        colsp = plsc.load_gather(cbuf, [esp])
        rowsp = plsc.load_gather(rbuf, [esp])
        dsp = plsc.load_gather(dbuf, [esp])
        for k in range(D // 16):
            yk = plsc.load_gather(yv, [colsp, fks[k]])
            hk = jnp.maximum(yk + dsp * uks[k], 0.0)
            plsc.addupdate_scatter(hloc, [rowsp, fks[k]], hk)

    pltpu.sync_copy(hloc, h_hbm.at[pl.ds(t * RPT, RPT)])


def kernel(src_tokens, padded_coordinates, src_distance, src_edge_type,
           embed_tokens, params):
    del src_distance, src_edge_type
    tok = src_tokens.astype(jnp.int32)
    coord = padded_coordinates.astype(jnp.float32).reshape(BN, 3)
    cpad = jnp.pad(coord, ((0, 0), (0, D - 3)))
    oh = (tok.reshape(BN, 1) == jnp.arange(V, dtype=jnp.int32)[None, :]
          ).astype(jnp.float32)
    embed = embed_tokens.astype(jnp.float32)

    def prep(p):
        return (p["W1"], p["b1"].reshape(1, D), p["be"].reshape(1, D),
                p["We"].reshape(1, D), p["W2"], p["b2"].reshape(1, D))

    w1_0, b1_0, be_0, we_0, _, _ = prep(params[0])
    c3 = coord.reshape(B, 1, N, 3)
    md, x, y, deg, u = _tc0(oh, cpad, tok.reshape(B, 1, N),
                            c3[..., 0], c3[..., 1], c3[..., 2], embed,
                            w1_0, b1_0, be_0, we_0)
    cols, rows, dists, counts = _sc_compact(md)

    for l in range(L):
        h = _sc_edge(y, u, cols, rows, dists, counts)
        _, _, _, _, w2, b2 = prep(params[l])
        if l + 1 < L:
            w1n, b1n, ben, wen, _, _ = prep(params[l + 1])
            x, y, u = _tc_mid(x, h, deg, w2, b2, w1n, b1n, ben, wen)
        else:
            out = _tc_fin(x, h, deg, w2, b2, oh)

    encoder_rep = out.reshape(B, N, D)
    padding_mask = src_tokens == PAD
    return (encoder_rep, padding_mask)


# speedup vs baseline: 1.0833x; 1.0296x over previous
import functools

import jax
import jax.numpy as jnp
from jax import lax
from jax.experimental import pallas as pl
from jax.experimental.pallas import tpu as pltpu
from jax.experimental.pallas import tpu_sc as plsc

B, N, D, L, V, PAD, R = 8, 512, 128, 4, 32, 0, 6.0
BN = B * N
NT = 32
RPT = BN // NT
CAP = 8192

_HI = jax.lax.Precision.HIGHEST


def _dgT(a, b):
    return jax.lax.dot_general(a, b, (((1,), (1,)), ((), ())), precision=_HI)


def _mm(a, b):
    return jax.lax.dot_general(a, b, (((1,), (0,)), ((), ())), precision=_HI)


def _tc0_body(oh_ref, cpad_ref, tok_ref, cx_ref, cy_ref, cz_ref, embed_ref,
              w1_ref, b1_ref, be_ref, we_ref,
              md_ref, x0_ref, y0_ref, deg_ref, u_ref):
    oh = oh_ref[...]
    C = cpad_ref[...]
    ones = jnp.ones((N, D), jnp.float32)
    cxr, cyr, czr = cx_ref[0], cy_ref[0], cz_ref[0]
    cxc, cyc, czc = C[:, 0:1], C[:, 1:2], C[:, 2:3]
    sqi = cxc * cxc + cyc * cyc + czc * czc
    sqj = cxr * cxr + cyr * cyr + czr * czr
    dots = jax.lax.dot_general(C, C, (((1,), (1,)), ((), ())),
                               precision=jax.lax.Precision.DEFAULT)
    d2 = sqi + sqj - 2.0 * dots
    ri = jax.lax.broadcasted_iota(jnp.int32, (N, N), 0)
    rj = jax.lax.broadcasted_iota(jnp.int32, (N, N), 1)
    vi = oh[:, 0:1] < 0.5
    vj = tok_ref[0] != PAD
    adj = (d2 < R * R) & (ri != rj) & vi & vj
    ddx = cxc - cxr
    ddy = cyc - cyr
    ddz = czc - czr
    dist = jnp.sqrt(ddx * ddx + ddy * ddy + ddz * ddz)
    md_ref[...] = jnp.where(adj, dist, -1.0)
    adjf = adj.astype(jnp.float32)
    deg_ref[...] = _mm(adjf, ones)
    x0 = _mm(oh, embed_ref[...])
    x0_ref[...] = x0
    A = w1_ref[0:D, :]
    Bm = w1_ref[D:2 * D, :]
    cvec = _mm(be_ref[...], Bm) + b1_ref[...]
    y0_ref[...] = _mm(x0, A) + cvec
    u_ref[...] = _mm(we_ref[...], Bm)


def _tc0(oh, cpad, tok, cx, cy, cz, embed, w1, b1, be, we):
    blk = lambda shape, imap: pl.BlockSpec(shape, imap)
    return pl.pallas_call(
        _tc0_body,
        grid=(B,),
        in_specs=[
            blk((N, V), lambda b: (b, 0)),
            blk((N, D), lambda b: (b, 0)),
            blk((1, 1, N), lambda b: (b, 0, 0)),
            blk((1, 1, N), lambda b: (b, 0, 0)),
            blk((1, 1, N), lambda b: (b, 0, 0)),
            blk((1, 1, N), lambda b: (b, 0, 0)),
            blk((V, D), lambda b: (0, 0)),
            blk((2 * D, D), lambda b: (0, 0)),
            blk((1, D), lambda b: (0, 0)),
            blk((1, D), lambda b: (0, 0)),
            blk((1, D), lambda b: (0, 0)),
        ],
        out_specs=[
            blk((N, N), lambda b: (b, 0)),
            blk((N, D), lambda b: (b, 0)),
            blk((N, D), lambda b: (b, 0)),
            blk((N, D), lambda b: (b, 0)),
            blk((1, D), lambda b: (0, 0)),
        ],
        out_shape=[
            jax.ShapeDtypeStruct((BN, N), jnp.float32),
            jax.ShapeDtypeStruct((BN, D), jnp.float32),
            jax.ShapeDtypeStruct((BN, D), jnp.float32),
            jax.ShapeDtypeStruct((BN, D), jnp.float32),
            jax.ShapeDtypeStruct((1, D), jnp.float32),
        ],
    )(oh, cpad, tok, cx, cy, cz, embed, w1, b1, be, we)


def _tc_mid_body(x_ref, h_ref, deg_ref, w2_ref, b2_ref, w1_ref, b1_ref,
                 be_ref, we_ref, xn_ref, y_ref, u_ref):
    xn = x_ref[...] + _mm(h_ref[...], w2_ref[...]) + deg_ref[...] * b2_ref[...]
    xn_ref[...] = xn
    A = w1_ref[0:D, :]
    Bm = w1_ref[D:2 * D, :]
    cvec = _mm(be_ref[...], Bm) + b1_ref[...]
    y_ref[...] = _mm(xn, A) + cvec
    u_ref[...] = _mm(we_ref[...], Bm)


def _tc_mid(x, h, deg, w2, b2, w1, b1, be, we):
    blk = lambda shape, imap: pl.BlockSpec(shape, imap)
    return pl.pallas_call(
        _tc_mid_body,
        grid=(B,),
        in_specs=[
            blk((N, D), lambda b: (b, 0)),
            blk((N, D), lambda b: (b, 0)),
            blk((N, D), lambda b: (b, 0)),
            blk((D, D), lambda b: (0, 0)),
            blk((1, D), lambda b: (0, 0)),
            blk((2 * D, D), lambda b: (0, 0)),
            blk((1, D), lambda b: (0, 0)),
            blk((1, D), lambda b: (0, 0)),
            blk((1, D), lambda b: (0, 0)),
        ],
        out_specs=[
            blk((N, D), lambda b: (b, 0)),
            blk((N, D), lambda b: (b, 0)),
            blk((1, D), lambda b: (0, 0)),
        ],
        out_shape=[
            jax.ShapeDtypeStruct((BN, D), jnp.float32),
            jax.ShapeDtypeStruct((BN, D), jnp.float32),
            jax.ShapeDtypeStruct((1, D), jnp.float32),
        ],
    )(x, h, deg, w2, b2, w1, b1, be, we)


def _tc_fin_body(x_ref, h_ref, deg_ref, w2_ref, b2_ref, oh_ref, out_ref):
    xn = x_ref[...] + _mm(h_ref[...], w2_ref[...]) + deg_ref[...] * b2_ref[...]
    vi = oh_ref[:, 0:1] < 0.5
    out_ref[...] = jnp.where(vi, xn, 0.0)


def _tc_fin(x, h, deg, w2, b2, oh):
    blk = lambda shape, imap: pl.BlockSpec(shape, imap)
    return pl.pallas_call(
        _tc_fin_body,
        grid=(B,),
        in_specs=[
            blk((N, D), lambda b: (b, 0)),
            blk((N, D), lambda b: (b, 0)),
            blk((N, D), lambda b: (b, 0)),
            blk((D, D), lambda b: (0, 0)),
            blk((1, D), lambda b: (0, 0)),
            blk((N, V), lambda b: (b, 0)),
        ],
        out_specs=[blk((N, D), lambda b: (b, 0))],
        out_shape=[jax.ShapeDtypeStruct((BN, D), jnp.float32)],
    )(x, h, deg, w2, b2, oh)[0]


_MESH = plsc.VectorSubcoreMesh(core_axis_name="c", subcore_axis_name="s")


def _iota16():
    return jax.lax.broadcasted_iota(jnp.int32, (16,), 0)


@functools.partial(
    pl.kernel,
    out_type=[
        jax.ShapeDtypeStruct((NT, CAP), jnp.int32),
        jax.ShapeDtypeStruct((NT, CAP), jnp.int32),
        jax.ShapeDtypeStruct((NT, CAP), jnp.float32),
        jax.ShapeDtypeStruct((NT, 16), jnp.int32),
    ],
    mesh=_MESH,
    compiler_params=pltpu.CompilerParams(needs_layout_passes=False),
    scratch_types=[
        pltpu.VMEM((RPT, N), jnp.float32),
        pltpu.VMEM((CAP,), jnp.int32),
        pltpu.VMEM((CAP,), jnp.int32),
        pltpu.VMEM((CAP,), jnp.float32),
        pltpu.VMEM((16,), jnp.int32),
    ],
)
def _sc_compact(md_hbm, cols_hbm, rows_hbm, dists_hbm, counts_hbm,
                mdv, colv, rowv, distv, cntv):
    c = lax.axis_index("c")
    s = lax.axis_index("s")
    t = c * 16 + s
    it16 = _iota16()
    pltpu.sync_copy(md_hbm.at[pl.ds(t * RPT, RPT)], mdv)

    def row_body(i, ptr):
        isp = jnp.full((16,), i, jnp.int32)

        @plsc.parallel_loop(0, N // 16, carry=ptr, unroll=8)
        def grp_body(g, ptr):
            jloc = g * 16 + it16
            v = plsc.load_gather(mdv, [isp, jloc])
            m = v >= 0.0
            pos = ptr + jnp.cumsum(m.astype(jnp.int32)) - 1
            mm = m & (pos < CAP)
            plsc.store_scatter(colv, [pos], jloc, mask=mm)
            plsc.store_scatter(rowv, [pos], isp, mask=mm)
            plsc.store_scatter(distv, [pos], v, mask=mm)
            return ptr + plsc.all_reduce_population_count(m)
        return grp_body

    ptr = lax.fori_loop(0, RPT, row_body, jnp.zeros((16,), jnp.int32))
    cntv[...] = jnp.minimum(ptr, CAP)
    pltpu.sync_copy(colv, cols_hbm.at[t])
    pltpu.sync_copy(rowv, rows_hbm.at[t])
    pltpu.sync_copy(distv, dists_hbm.at[t])
    pltpu.sync_copy(cntv, counts_hbm.at[t])


@functools.partial(
    pl.kernel,
    out_type=jax.ShapeDtypeStruct((BN, D), jnp.float32),
    mesh=_MESH,
    compiler_params=pltpu.CompilerParams(needs_layout_passes=False),
    scratch_types=[
        pltpu.VMEM((N, D), jnp.float32),
        pltpu.VMEM((RPT, D), jnp.float32),
        pltpu.VMEM((CAP,), jnp.int32),
        pltpu.VMEM((CAP,), jnp.int32),
        pltpu.VMEM((CAP,), jnp.float32),
        pltpu.VMEM((16,), jnp.int32),
        pltpu.VMEM((1, D), jnp.float32),
    ],
)
def _sc_edge(y_hbm, u_hbm, cols_hbm, rows_hbm, dists_hbm, counts_hbm,
             h_hbm, yv, hloc, cbuf, rbuf, dbuf, cntv, uv):
    c = lax.axis_index("c")
    s = lax.axis_index("s")
    t = c * 16 + s
    b = t >> 2
    it16 = _iota16()
    z16 = jnp.zeros((16,), jnp.int32)
    z16f = jnp.zeros((16,), jnp.float32)
    pltpu.sync_copy(y_hbm.at[pl.ds(b * N, N)], yv)
    pltpu.sync_copy(u_hbm, uv)
    pltpu.sync_copy(cols_hbm.at[t], cbuf)
    pltpu.sync_copy(rows_hbm.at[t], rbuf)
    pltpu.sync_copy(dists_hbm.at[t], dbuf)
    pltpu.sync_copy(counts_hbm.at[t], cntv)
    count = jnp.max(cntv[...])

    uks = [plsc.load_gather(uv, [z16, k * 16 + it16])
           for k in range(D // 16)]
    fks = [k * 16 + it16 for k in range(D // 16)]

    @plsc.parallel_loop(0, RPT)
    def zero_body(r):
        rsp = jnp.full((16,), r, jnp.int32)
        for k in range(D // 16):
            plsc.store_scatter(hloc, [rsp, fks[k]], z16f)

    @plsc.parallel_loop(0, count, unroll=2)
    def e_body(e):
        esp = jnp.full((16,), e, jnp.int32)
        colsp = plsc.load_gather(cbuf, [esp])
        rowsp = plsc.load_gather(rbuf, [esp])
        dsp = plsc.load_gather(dbuf, [esp])
        for k in range(D // 16):
            yk = plsc.load_gather(yv, [colsp, fks[k]])
            hk = jnp.maximum(yk + dsp * uks[k], 0.0)
            plsc.addupdate_scatter(hloc, [rowsp, fks[k]], hk)

    pltpu.sync_copy(hloc, h_hbm.at[pl.ds(t * RPT, RPT)])


def kernel(src_tokens, padded_coordinates, src_distance, src_edge_type,
           embed_tokens, params):
    del src_distance, src_edge_type
    tok = src_tokens.astype(jnp.int32)
    coord = padded_coordinates.astype(jnp.float32).reshape(BN, 3)
    cpad = jnp.pad(coord, ((0, 0), (0, D - 3)))
    oh = (tok.reshape(BN, 1) == jnp.arange(V, dtype=jnp.int32)[None, :]
          ).astype(jnp.float32)
    embed = embed_tokens.astype(jnp.float32)

    def prep(p):
        return (p["W1"], p["b1"].reshape(1, D), p["be"].reshape(1, D),
                p["We"].reshape(1, D), p["W2"], p["b2"].reshape(1, D))

    w1_0, b1_0, be_0, we_0, _, _ = prep(params[0])
    c3 = coord.reshape(B, 1, N, 3)
    md, x, y, deg, u = _tc0(oh, cpad, tok.reshape(B, 1, N),
                            c3[..., 0], c3[..., 1], c3[..., 2], embed,
                            w1_0, b1_0, be_0, we_0)
    cols, rows, dists, counts = _sc_compact(md)

    for l in range(L):
        h = _sc_edge(y, u, cols, rows, dists, counts)
        _, _, _, _, w2, b2 = prep(params[l])
        if l + 1 < L:
            w1n, b1n, ben, wen, _, _ = prep(params[l + 1])
            x, y, u = _tc_mid(x, h, deg, w2, b2, w1n, b1n, ben, wen)
        else:
            out = _tc_fin(x, h, deg, w2, b2, oh)

    encoder_rep = out.reshape(B, N, D)
    padding_mask = src_tokens == PAD
    return (encoder_rep, padding_mask)


# speedup vs baseline: 1.0856x; 1.0022x over previous
import functools

import jax
import jax.numpy as jnp
from jax import lax
from jax.experimental import pallas as pl
from jax.experimental.pallas import tpu as pltpu
from jax.experimental.pallas import tpu_sc as plsc

B, N, D, L, V, PAD, R = 8, 512, 128, 4, 32, 0, 6.0
BN = B * N
NT = 32
RPT = BN // NT
CAP = 8192

_HI = jax.lax.Precision.HIGHEST


def _mm(a, b):
    return jax.lax.dot_general(a, b, (((1,), (0,)), ((), ())), precision=_HI)


def _tc0_body(oh_ref, cpad_ref, tok_ref, cx_ref, cy_ref, cz_ref, embed_ref,
              w1_ref, b1_ref, be_ref, we_ref,
              md_ref, x0_ref, y0_ref, deg_ref, u_ref):
    oh = oh_ref[...]
    C = cpad_ref[...]
    ones = jnp.ones((N, D), jnp.float32)
    cxr, cyr, czr = cx_ref[0], cy_ref[0], cz_ref[0]
    cxc, cyc, czc = C[:, 0:1], C[:, 1:2], C[:, 2:3]
    sqi = cxc * cxc + cyc * cyc + czc * czc
    sqj = cxr * cxr + cyr * cyr + czr * czr
    dots = jax.lax.dot_general(C, C, (((1,), (1,)), ((), ())),
                               precision=jax.lax.Precision.DEFAULT)
    d2 = sqi + sqj - 2.0 * dots
    ri = jax.lax.broadcasted_iota(jnp.int32, (N, N), 0)
    rj = jax.lax.broadcasted_iota(jnp.int32, (N, N), 1)
    vi = oh[:, 0:1] < 0.5
    vj = tok_ref[0] != PAD
    adj = (d2 < R * R) & (ri != rj) & vi & vj
    ddx = cxc - cxr
    ddy = cyc - cyr
    ddz = czc - czr
    dist = jnp.sqrt(ddx * ddx + ddy * ddy + ddz * ddz)
    md_ref[...] = jnp.where(adj, dist, -1.0)
    adjf = adj.astype(jnp.float32)
    deg_ref[...] = _mm(adjf, ones)
    x0 = _mm(oh, embed_ref[...])
    x0_ref[...] = x0
    A = w1_ref[0:D, :]
    Bm = w1_ref[D:2 * D, :]
    cvec = _mm(be_ref[...], Bm) + b1_ref[...]
    y0_ref[...] = _mm(x0, A) + cvec
    u_ref[...] = _mm(we_ref[...], Bm)


def _tc0(oh, cpad, tok, cx, cy, cz, embed, w1, b1, be, we):
    blk = lambda shape, imap: pl.BlockSpec(shape, imap)
    return pl.pallas_call(
        _tc0_body,
        grid=(B,),
        in_specs=[
            blk((N, V), lambda b: (b, 0)),
            blk((N, D), lambda b: (b, 0)),
            blk((1, 1, N), lambda b: (b, 0, 0)),
            blk((1, 1, N), lambda b: (b, 0, 0)),
            blk((1, 1, N), lambda b: (b, 0, 0)),
            blk((1, 1, N), lambda b: (b, 0, 0)),
            blk((V, D), lambda b: (0, 0)),
            blk((2 * D, D), lambda b: (0, 0)),
            blk((1, D), lambda b: (0, 0)),
            blk((1, D), lambda b: (0, 0)),
            blk((1, D), lambda b: (0, 0)),
        ],
        out_specs=[
            blk((N, N), lambda b: (b, 0)),
            blk((N, D), lambda b: (b, 0)),
            blk((N, D), lambda b: (b, 0)),
            blk((N, D), lambda b: (b, 0)),
            blk((1, D), lambda b: (0, 0)),
        ],
        out_shape=[
            jax.ShapeDtypeStruct((BN, N), jnp.float32),
            jax.ShapeDtypeStruct((BN, D), jnp.float32),
            jax.ShapeDtypeStruct((BN, D), jnp.float32),
            jax.ShapeDtypeStruct((BN, D), jnp.float32),
            jax.ShapeDtypeStruct((1, D), jnp.float32),
        ],
    )(oh, cpad, tok, cx, cy, cz, embed, w1, b1, be, we)


def _tc_mid_body(x_ref, h_ref, deg_ref, w2_ref, b2_ref, w1_ref, b1_ref,
                 be_ref, we_ref, xn_ref, y_ref, u_ref):
    xn = x_ref[...] + _mm(h_ref[...], w2_ref[...]) + deg_ref[...] * b2_ref[...]
    xn_ref[...] = xn
    A = w1_ref[0:D, :]
    Bm = w1_ref[D:2 * D, :]
    cvec = _mm(be_ref[...], Bm) + b1_ref[...]
    y_ref[...] = _mm(xn, A) + cvec
    u_ref[...] = _mm(we_ref[...], Bm)


def _tc_mid(x, h, deg, w2, b2, w1, b1, be, we):
    blk = lambda shape, imap: pl.BlockSpec(shape, imap)
    return pl.pallas_call(
        _tc_mid_body,
        grid=(B,),
        in_specs=[
            blk((N, D), lambda b: (b, 0)),
            blk((N, D), lambda b: (b, 0)),
            blk((N, D), lambda b: (b, 0)),
            blk((D, D), lambda b: (0, 0)),
            blk((1, D), lambda b: (0, 0)),
            blk((2 * D, D), lambda b: (0, 0)),
            blk((1, D), lambda b: (0, 0)),
            blk((1, D), lambda b: (0, 0)),
            blk((1, D), lambda b: (0, 0)),
        ],
        out_specs=[
            blk((N, D), lambda b: (b, 0)),
            blk((N, D), lambda b: (b, 0)),
            blk((1, D), lambda b: (0, 0)),
        ],
        out_shape=[
            jax.ShapeDtypeStruct((BN, D), jnp.float32),
            jax.ShapeDtypeStruct((BN, D), jnp.float32),
            jax.ShapeDtypeStruct((1, D), jnp.float32),
        ],
    )(x, h, deg, w2, b2, w1, b1, be, we)


def _tc_fin_body(x_ref, h_ref, deg_ref, w2_ref, b2_ref, oh_ref, out_ref):
    xn = x_ref[...] + _mm(h_ref[...], w2_ref[...]) + deg_ref[...] * b2_ref[...]
    vi = oh_ref[:, 0:1] < 0.5
    out_ref[...] = jnp.where(vi, xn, 0.0)


def _tc_fin(x, h, deg, w2, b2, oh):
    blk = lambda shape, imap: pl.BlockSpec(shape, imap)
    return pl.pallas_call(
        _tc_fin_body,
        grid=(B,),
        in_specs=[
            blk((N, D), lambda b: (b, 0)),
            blk((N, D), lambda b: (b, 0)),
            blk((N, D), lambda b: (b, 0)),
            blk((D, D), lambda b: (0, 0)),
            blk((1, D), lambda b: (0, 0)),
            blk((N, V), lambda b: (b, 0)),
        ],
        out_specs=[blk((N, D), lambda b: (b, 0))],
        out_shape=[jax.ShapeDtypeStruct((BN, D), jnp.float32)],
    )(x, h, deg, w2, b2, oh)[0]


_MESH = plsc.VectorSubcoreMesh(core_axis_name="c", subcore_axis_name="s")


def _iota16():
    return jax.lax.broadcasted_iota(jnp.int32, (16,), 0)


@functools.partial(
    pl.kernel,
    out_type=[
        jax.ShapeDtypeStruct((NT, CAP), jnp.int32),
        jax.ShapeDtypeStruct((NT, CAP), jnp.int32),
        jax.ShapeDtypeStruct((NT, CAP), jnp.float32),
        jax.ShapeDtypeStruct((NT, 16), jnp.int32),
    ],
    mesh=_MESH,
    compiler_params=pltpu.CompilerParams(needs_layout_passes=False),
    scratch_types=[
        pltpu.VMEM((RPT, N), jnp.float32),
        pltpu.VMEM((CAP,), jnp.int32),
        pltpu.VMEM((CAP,), jnp.int32),
        pltpu.VMEM((CAP,), jnp.float32),
        pltpu.VMEM((16,), jnp.int32),
    ],
)
def _sc_compact(md_hbm, cols_hbm, rows_hbm, dists_hbm, counts_hbm,
                mdv, colv, rowv, distv, cntv):
    c = lax.axis_index("c")
    s = lax.axis_index("s")
    t = c * 16 + s
    it16 = _iota16()
    pltpu.sync_copy(md_hbm.at[pl.ds(t * RPT, RPT)], mdv)

    def row_body(i, ptr):
        isp = jnp.full((16,), i, jnp.int32)

        @plsc.parallel_loop(0, N // 16, carry=ptr, unroll=8)
        def grp_body(g, ptr):
            jloc = g * 16 + it16
            v = plsc.load_gather(mdv, [isp, jloc])
            m = v >= 0.0
            pos = ptr + jnp.cumsum(m.astype(jnp.int32)) - 1
            mm = m & (pos < CAP)
            plsc.store_scatter(colv, [pos], jloc, mask=mm)
            plsc.store_scatter(rowv, [pos], isp, mask=mm)
            plsc.store_scatter(distv, [pos], v, mask=mm)
            return ptr + plsc.all_reduce_population_count(m)
        return grp_body

    ptr = lax.fori_loop(0, RPT, row_body, jnp.zeros((16,), jnp.int32))
    cntv[...] = jnp.minimum(ptr, CAP)
    pltpu.sync_copy(colv, cols_hbm.at[t])
    pltpu.sync_copy(rowv, rows_hbm.at[t])
    pltpu.sync_copy(distv, dists_hbm.at[t])
    pltpu.sync_copy(cntv, counts_hbm.at[t])


@functools.partial(
    pl.kernel,
    out_type=jax.ShapeDtypeStruct((BN, D), jnp.float32),
    mesh=_MESH,
    compiler_params=pltpu.CompilerParams(needs_layout_passes=False),
    scratch_types=[
        pltpu.VMEM((N, D), jnp.float32),
        pltpu.VMEM((RPT, D), jnp.float32),
        pltpu.VMEM((CAP,), jnp.int32),
        pltpu.VMEM((CAP,), jnp.int32),
        pltpu.VMEM((CAP,), jnp.float32),
        pltpu.VMEM((16,), jnp.int32),
        pltpu.VMEM((1, D), jnp.float32),
    ],
)
def _sc_edge(y_hbm, u_hbm, cols_hbm, rows_hbm, dists_hbm, counts_hbm,
             h_hbm, yv, hloc, cbuf, rbuf, dbuf, cntv, uv):
    c = lax.axis_index("c")
    s = lax.axis_index("s")
    t = c * 16 + s
    b = t >> 2
    it16 = _iota16()
    z16 = jnp.zeros((16,), jnp.int32)
    z16f = jnp.zeros((16,), jnp.float32)
    pltpu.sync_copy(y_hbm.at[pl.ds(b * N, N)], yv)
    pltpu.sync_copy(u_hbm, uv)
    pltpu.sync_copy(cols_hbm.at[t], cbuf)
    pltpu.sync_copy(rows_hbm.at[t], rbuf)
    pltpu.sync_copy(dists_hbm.at[t], dbuf)
    pltpu.sync_copy(counts_hbm.at[t], cntv)
    count = jnp.max(cntv[...])

    uks = [plsc.load_gather(uv, [z16, k * 16 + it16])
           for k in range(D // 16)]
    fks = [k * 16 + it16 for k in range(D // 16)]

    @plsc.parallel_loop(0, RPT)
    def zero_body(r):
        rsp = jnp.full((16,), r, jnp.int32)
        for k in range(D // 16):
            plsc.store_scatter(hloc, [rsp, fks[k]], z16f)

    @plsc.parallel_loop(0, count, unroll=2)
    def e_body(e):
        esp = jnp.full((16,), e, jnp.int32)
        colsp = plsc.load_gather(cbuf, [esp])
        rowsp = plsc.load_gather(rbuf, [esp])
        dsp = plsc.load_gather(dbuf, [esp])
        for k in range(D // 16):
            yk = plsc.load_gather(yv, [colsp, fks[k]])
            hk = jnp.maximum(yk + dsp * uks[k], 0.0)
            plsc.addupdate_scatter(hloc, [rowsp, fks[k]], hk)

    pltpu.sync_copy(hloc, h_hbm.at[pl.ds(t * RPT, RPT)])


def kernel(src_tokens, padded_coordinates, src_distance, src_edge_type,
           embed_tokens, params):
    del src_distance, src_edge_type
    tok = src_tokens.astype(jnp.int32)
    coord = padded_coordinates.astype(jnp.float32).reshape(BN, 3)
    cpad = jnp.pad(coord, ((0, 0), (0, D - 3)))
    oh = (tok.reshape(BN, 1) == jnp.arange(V, dtype=jnp.int32)[None, :]
          ).astype(jnp.float32)
    embed = embed_tokens.astype(jnp.float32)

    def prep(p):
        return (p["W1"], p["b1"].reshape(1, D), p["be"].reshape(1, D),
                p["We"].reshape(1, D), p["W2"], p["b2"].reshape(1, D))

    w1_0, b1_0, be_0, we_0, _, _ = prep(params[0])
    c3 = coord.reshape(B, 1, N, 3)
    md, x, y, deg, u = _tc0(oh, cpad, tok.reshape(B, 1, N),
                            c3[..., 0], c3[..., 1], c3[..., 2], embed,
                            w1_0, b1_0, be_0, we_0)
    cols, rows, dists, counts = _sc_compact(md)

    for l in range(L):
        h = _sc_edge(y, u, cols, rows, dists, counts)
        _, _, _, _, w2, b2 = prep(params[l])
        if l + 1 < L:
            w1n, b1n, ben, wen, _, _ = prep(params[l + 1])
            x, y, u = _tc_mid(x, h, deg, w2, b2, w1n, b1n, ben, wen)
        else:
            out = _tc_fin(x, h, deg, w2, b2, oh)

    encoder_rep = out.reshape(B, N, D)
    padding_mask = src_tokens == PAD
    return (encoder_rep, padding_mask)
